# baseline (speedup 1.0000x reference)
import functools
import jax
import jax.numpy as jnp
from jax import lax
from jax.experimental import pallas as pl
from jax.experimental.pallas import tpu as pltpu
from jax.experimental.pallas import tpu_sc as plsc

N = 4096
H = 128
NRBF = 50
NRBF_PAD = 64
CUTOFF = 0.2
L = 4
VMAX = 100
VMAX_PAD = 128
NM = 128
OUT = 1
K = 256
PADV = -1e4
BI = 16


def _emb_body(an_ref, emb_ref, out_ref):
    an = an_ref[...]
    ids = lax.broadcasted_iota(jnp.int32, (1, VMAX_PAD), 1)
    onehot = (jnp.clip(an, 0, VMAX - 1) == ids).astype(jnp.float32)
    out_ref[...] = jnp.dot(onehot, emb_ref[...], preferred_element_type=jnp.float32, precision=lax.Precision.HIGHEST)


def _embed(an, emb_pad, interpret=False):
    return pl.pallas_call(
        _emb_body,
        out_shape=jax.ShapeDtypeStruct((N, H), jnp.float32),
        interpret=interpret,
    )(an.reshape(N, 1).astype(jnp.int32), emb_pad)


def _pre_body(x_ref, w_ref, b_ref, out_ref):
    pre = jnp.dot(x_ref[...], w_ref[...], preferred_element_type=jnp.float32) + b_ref[...]
    out_ref[:N, :] = pre
    out_ref[N:, :] = jnp.full((8, H), PADV, jnp.float32)


def _pre(x, W1x, b1, interpret=False):
    return pl.pallas_call(
        _pre_body,
        out_shape=jax.ShapeDtypeStruct((N + 8, H), jnp.float32),
        interpret=interpret,
    )(x, W1x, b1.reshape(1, H))


def _msg_body(cen_ref, w1r_ref, w2_ref, b2_ref, wih_ref, bih_ref, bhh_ref,
              g_ref, d2_ref, vm_ref, deg_ref, x_ref, out_ref):
    width = CUTOFF / NRBF
    d = jnp.sqrt(d2_ref[...])
    rbf = jnp.exp(-((d - cen_ref[...]) ** 2) / (2.0 * width * width))
    rbfw = jnp.dot(rbf, w1r_ref[...], preferred_element_type=jnp.float32)
    z = g_ref[...] + rbfw
    m = z * lax.logistic(z) * vm_ref[...]
    y = jnp.dot(m, w2_ref[...], preferred_element_type=jnp.float32)
    aggr = y.reshape(BI, K, H).sum(axis=1)
    aggr = aggr + deg_ref[...] * b2_ref[...]
    gi = jnp.dot(aggr, wih_ref[...], preferred_element_type=jnp.float32) + bih_ref[...]
    bhh = bhh_ref[...]
    r = lax.logistic(gi[:, :H] + bhh[:, :H])
    zz = lax.logistic(gi[:, H:2 * H] + bhh[:, H:2 * H])
    n = jnp.tanh(gi[:, 2 * H:] + r * bhh[:, 2 * H:])
    out_ref[...] = x_ref[...] + (1.0 - zz) * n


def _msg_layer(centers, W1r, W2, b2, Wih, bih, bhh, G, d2e, vmask, deg, x, interpret=False):
    nblk = N // BI
    grid = (nblk,)
    return pl.pallas_call(
        _msg_body,
        grid=grid,
        in_specs=[
            pl.BlockSpec((1, NRBF_PAD), lambda i: (0, 0)),
            pl.BlockSpec((NRBF_PAD, H), lambda i: (0, 0)),
            pl.BlockSpec((H, H), lambda i: (0, 0)),
            pl.BlockSpec((1, H), lambda i: (0, 0)),
            pl.BlockSpec((H, 3 * H), lambda i: (0, 0)),
            pl.BlockSpec((1, 3 * H), lambda i: (0, 0)),
            pl.BlockSpec((1, 3 * H), lambda i: (0, 0)),
            pl.BlockSpec((BI * K, H), lambda i: (i, 0)),
            pl.BlockSpec((BI * K, 1), lambda i: (i, 0)),
            pl.BlockSpec((BI * K, 1), lambda i: (i, 0)),
            pl.BlockSpec((BI, 1), lambda i: (i, 0)),
            pl.BlockSpec((BI, H), lambda i: (i, 0)),
        ],
        out_specs=pl.BlockSpec((BI, H), lambda i: (i, 0)),
        out_shape=jax.ShapeDtypeStruct((N, H), jnp.float32),
        interpret=interpret,
    )(centers, W1r, W2, b2.reshape(1, H), Wih, bih.reshape(1, 3 * H),
      bhh.reshape(1, 3 * H), G, d2e.reshape(N * K, 1), vmask.reshape(N * K, 1), deg, x)


def _head_body(x_ref, batch_ref, g_ref, b_ref, w1_ref, b1_ref, w2_ref, b2_ref, out_ref):
    x = x_ref[...]
    mu = jnp.mean(x, axis=1, keepdims=True)
    xc = x - mu
    var = jnp.mean(xc * xc, axis=1, keepdims=True)
    xh = xc * lax.rsqrt(var + 1e-5) * g_ref[...] + b_ref[...]
    mids = lax.broadcasted_iota(jnp.int32, (NM, 1), 0)
    sel = (mids == batch_ref[...]).astype(jnp.float32)
    graph = jnp.dot(sel, xh, preferred_element_type=jnp.float32, precision=lax.Precision.HIGHEST)
    counts = jnp.sum(sel, axis=1, keepdims=True)
    graph = graph / jnp.maximum(counts, 1.0)
    h1 = jnp.dot(graph, w1_ref[...], preferred_element_type=jnp.float32) + b1_ref[...]
    h1 = h1 * lax.logistic(h1)
    out_ref[...] = jnp.dot(h1, w2_ref[...], preferred_element_type=jnp.float32) + b2_ref[...]


def _head(x, batch, ln_g, ln_b, hW1, hb1, hW2, hb2, interpret=False):
    return pl.pallas_call(
        _head_body,
        out_shape=jax.ShapeDtypeStruct((NM, OUT), jnp.float32),
        interpret=interpret,
    )(x, batch.reshape(1, N).astype(jnp.int32), ln_g.reshape(1, H), ln_b.reshape(1, H),
      hW1, hb1.reshape(1, H), hW2, hb2.reshape(1, OUT))


def _extract_stub(positions):
    diff = positions[:, None, :] - positions[None, :, :]
    d2 = (diff * diff).sum(-1)
    mask = (d2 < CUTOFF * CUTOFF) & (~jnp.eye(N, dtype=bool))
    deg = mask.sum(1)
    keys = jnp.where(mask, -jnp.arange(N, dtype=jnp.int32), jnp.int32(-2 * N))
    _, nbr = lax.top_k(keys, K)
    slot_valid = jnp.arange(K)[None, :] < deg[:, None]
    own = jnp.broadcast_to(jnp.arange(N)[:, None], (N, K))
    nbr = jnp.where(slot_valid, nbr, own)
    d2e = jnp.where(slot_valid, jnp.take_along_axis(d2, nbr, axis=1), 1e6)
    vmask = slot_valid.astype(jnp.float32)
    return (nbr.astype(jnp.int32), d2e, vmask,
            deg.astype(jnp.float32).reshape(N, 1))


def _gather_stub(pre_pad, nbr):
    return pre_pad[nbr.reshape(-1)]


NC = 2
NS = 16
NW = NC * NS
GROWS = N * K // NW
GSUB = 128
GITER_SUB = 4
GCHUNK = GSUB * GITER_SUB


def _sc_gather(pre_pad, nbr2):
    mesh = plsc.VectorSubcoreMesh(core_axis_name="c", subcore_axis_name="s",
                                  num_cores=NC, num_subcores=NS)

    half = 2 * GSUB
    nsteps = GROWS // half

    @functools.partial(
        pl.kernel,
        out_type=jax.ShapeDtypeStruct((N * K, H), jnp.float32),
        mesh=mesh,
        scratch_types=[
            pltpu.VMEM((2, GSUB), jnp.int32),
            pltpu.VMEM((2, GSUB), jnp.int32),
            pltpu.VMEM((half, H), jnp.float32),
            pltpu.VMEM((half, H), jnp.float32),
            pltpu.SemaphoreType.DMA,
            pltpu.SemaphoreType.DMA,
            pltpu.SemaphoreType.DMA,
        ],
    )
    def k(pre_hbm, nbr_hbm, out_hbm, idx_a, idx_b, rows_a, rows_b, gsem, wsem_a, wsem_b):
        wid = lax.axis_index("s") * NC + lax.axis_index("c")
        nbr_row0 = wid * (GROWS // GSUB)
        out_row0 = wid * GROWS

        def body(i, carry):
            for h, (idxv, rowsv, wsem) in enumerate(
                    ((idx_a, rows_a, wsem_a), (idx_b, rows_b, wsem_b))):
                s = i * 2 + h
                @pl.when(s >= 2)
                def _():
                    pltpu.make_async_copy(out_hbm.at[pl.ds(0, half)], rowsv, wsem).wait()
                pltpu.sync_copy(nbr_hbm.at[pl.ds(nbr_row0 + s * 2, 2)], idxv)
                c0 = pltpu.async_copy(pre_hbm.at[idxv.at[0]],
                                      rowsv.at[pl.ds(0, GSUB)], gsem)
                c1 = pltpu.async_copy(pre_hbm.at[idxv.at[1]],
                                      rowsv.at[pl.ds(GSUB, GSUB)], gsem)
                c0.wait()
                c1.wait()
                pltpu.async_copy(rowsv, out_hbm.at[pl.ds(out_row0 + s * half, half)],
                                 wsem)
            return carry

        lax.fori_loop(0, nsteps // 2, body, 0)
        pltpu.make_async_copy(out_hbm.at[pl.ds(0, half)], rows_a, wsem_a).wait()
        pltpu.make_async_copy(out_hbm.at[pl.ds(0, half)], rows_b, wsem_b).wait()

    return k(pre_pad, nbr2)


def kernel(atomic_numbers, positions, batch, emb, msg_W1, msg_b1, msg_W2, msg_b2,
           gru_Wih, gru_Whh, gru_bih, gru_bhh, ln_g, ln_b,
           head_W1, head_b1, head_W2, head_b2, interpret=False):
    emb_pad = jnp.zeros((VMAX_PAD, H), jnp.float32).at[:VMAX].set(emb)
    centers = jnp.full((NRBF_PAD,), 1e6, jnp.float32).at[:NRBF].set(
        jnp.linspace(0.0, CUTOFF, NRBF)).reshape(1, NRBF_PAD)
    W1r_pad = jnp.zeros((L, NRBF_PAD, H), jnp.float32).at[:, :NRBF].set(msg_W1[:, H:])

    x = _embed(atomic_numbers, emb_pad, interpret)
    nbr, d2e, vmask, deg = _extract_stub(positions)
    nbr2 = nbr.reshape(N * K // GSUB, GSUB)
    for l in range(L):
        pre_pad = _pre(x, msg_W1[l, :H], msg_b1[l], interpret)
        if interpret:
            G = _gather_stub(pre_pad, nbr)
        else:
            G = _sc_gather(pre_pad, nbr2)
        x = _msg_layer(centers, W1r_pad[l], msg_W2[l], msg_b2[l],
                       gru_Wih[l], gru_bih[l], gru_bhh[l], G, d2e, vmask, deg, x, interpret)
    return _head(x, batch, ln_g, ln_b, head_W1, head_b1, head_W2, head_b2, interpret)


# speedup vs baseline: 1.0179x; 1.0179x over previous
import functools
import jax
import jax.numpy as jnp
from jax import lax
from jax.experimental import pallas as pl
from jax.experimental.pallas import tpu as pltpu
from jax.experimental.pallas import tpu_sc as plsc

N = 4096
H = 128
NRBF = 50
NRBF_PAD = 64
CUTOFF = 0.2
L = 4
VMAX = 100
VMAX_PAD = 128
NM = 128
OUT = 1
K = 256
PADV = -1e4
BI = 16


def _emb_body(an_ref, emb_ref, out_ref):
    an = an_ref[...]
    ids = lax.broadcasted_iota(jnp.int32, (1, VMAX_PAD), 1)
    onehot = (jnp.clip(an, 0, VMAX - 1) == ids).astype(jnp.float32)
    out_ref[...] = jnp.dot(onehot, emb_ref[...], preferred_element_type=jnp.float32, precision=lax.Precision.HIGHEST)


def _embed(an, emb_pad):
    return pl.pallas_call(
        _emb_body,
        out_shape=jax.ShapeDtypeStruct((N, H), jnp.float32),
    )(an.reshape(N, 1).astype(jnp.int32), emb_pad)


def _pre_body(x_ref, w_ref, b_ref, out_ref):
    pre = jnp.dot(x_ref[...], w_ref[...], preferred_element_type=jnp.float32) + b_ref[...]
    out_ref[:N, :] = pre
    out_ref[N:, :] = jnp.full((8, H), PADV, jnp.float32)


def _pre(x, W1x, b1):
    return pl.pallas_call(
        _pre_body,
        out_shape=jax.ShapeDtypeStruct((N + 8, H), jnp.float32),
    )(x, W1x, b1.reshape(1, H))


def _msg_body(cen_ref, w1r_ref, w2_ref, b2_ref, wih_ref, bih_ref, bhh_ref,
              g_ref, d2_ref, vm_ref, deg_ref, x_ref, out_ref):
    width = CUTOFF / NRBF
    d = jnp.sqrt(d2_ref[...])
    rbf = jnp.exp(-((d - cen_ref[...]) ** 2) / (2.0 * width * width))
    rbfw = jnp.dot(rbf, w1r_ref[...], preferred_element_type=jnp.float32)
    z = g_ref[...] + rbfw
    m = z * lax.logistic(z) * vm_ref[...]
    y = jnp.dot(m, w2_ref[...], preferred_element_type=jnp.float32)
    aggr = y.reshape(BI, K, H).sum(axis=1)
    aggr = aggr + deg_ref[...] * b2_ref[...]
    gi = jnp.dot(aggr, wih_ref[...], preferred_element_type=jnp.float32) + bih_ref[...]
    bhh = bhh_ref[...]
    r = lax.logistic(gi[:, :H] + bhh[:, :H])
    zz = lax.logistic(gi[:, H:2 * H] + bhh[:, H:2 * H])
    n = jnp.tanh(gi[:, 2 * H:] + r * bhh[:, 2 * H:])
    out_ref[...] = x_ref[...] + (1.0 - zz) * n


def _msg_layer(centers, W1r, W2, b2, Wih, bih, bhh, G, d2e, vmask, deg, x):
    nblk = N // BI
    grid = (nblk,)
    return pl.pallas_call(
        _msg_body,
        grid=grid,
        in_specs=[
            pl.BlockSpec((1, NRBF_PAD), lambda i: (0, 0)),
            pl.BlockSpec((NRBF_PAD, H), lambda i: (0, 0)),
            pl.BlockSpec((H, H), lambda i: (0, 0)),
            pl.BlockSpec((1, H), lambda i: (0, 0)),
            pl.BlockSpec((H, 3 * H), lambda i: (0, 0)),
            pl.BlockSpec((1, 3 * H), lambda i: (0, 0)),
            pl.BlockSpec((1, 3 * H), lambda i: (0, 0)),
            pl.BlockSpec((BI * K, H), lambda i: (i, 0)),
            pl.BlockSpec((BI * K, 1), lambda i: (i, 0)),
            pl.BlockSpec((BI * K, 1), lambda i: (i, 0)),
            pl.BlockSpec((BI, 1), lambda i: (i, 0)),
            pl.BlockSpec((BI, H), lambda i: (i, 0)),
        ],
        out_specs=pl.BlockSpec((BI, H), lambda i: (i, 0)),
        out_shape=jax.ShapeDtypeStruct((N, H), jnp.float32),
    )(centers, W1r, W2, b2.reshape(1, H), Wih, bih.reshape(1, 3 * H),
      bhh.reshape(1, 3 * H), G, d2e.reshape(N * K, 1), vmask.reshape(N * K, 1), deg, x)


def _head_body(x_ref, batch_ref, g_ref, b_ref, w1_ref, b1_ref, w2_ref, b2_ref, out_ref):
    x = x_ref[...]
    mu = jnp.mean(x, axis=1, keepdims=True)
    xc = x - mu
    var = jnp.mean(xc * xc, axis=1, keepdims=True)
    xh = xc * lax.rsqrt(var + 1e-5) * g_ref[...] + b_ref[...]
    mids = lax.broadcasted_iota(jnp.int32, (NM, 1), 0)
    sel = (mids == batch_ref[...]).astype(jnp.float32)
    graph = jnp.dot(sel, xh, preferred_element_type=jnp.float32, precision=lax.Precision.HIGHEST)
    counts = jnp.sum(sel, axis=1, keepdims=True)
    graph = graph / jnp.maximum(counts, 1.0)
    h1 = jnp.dot(graph, w1_ref[...], preferred_element_type=jnp.float32) + b1_ref[...]
    h1 = h1 * lax.logistic(h1)
    out_ref[...] = jnp.dot(h1, w2_ref[...], preferred_element_type=jnp.float32) + b2_ref[...]


def _head(x, batch, ln_g, ln_b, hW1, hb1, hW2, hb2):
    return pl.pallas_call(
        _head_body,
        out_shape=jax.ShapeDtypeStruct((NM, OUT), jnp.float32),
    )(x, batch.reshape(1, N).astype(jnp.int32), ln_g.reshape(1, H), ln_b.reshape(1, H),
      hW1, hb1.reshape(1, H), hW2, hb2.reshape(1, OUT))


def _neighbor_lists(positions):
    diff = positions[:, None, :] - positions[None, :, :]
    d2 = (diff * diff).sum(-1)
    mask = (d2 < CUTOFF * CUTOFF) & (~jnp.eye(N, dtype=bool))
    deg = mask.sum(1)
    order = jnp.argsort(~mask, axis=1, stable=True)
    nbr = order[:, :K]
    slot_valid = jnp.arange(K)[None, :] < deg[:, None]
    own = jnp.broadcast_to(jnp.arange(N)[:, None], (N, K))
    nbr = jnp.where(slot_valid, nbr, own)
    d2e = jnp.where(slot_valid, jnp.take_along_axis(d2, nbr, axis=1), 1e6)
    vmask = slot_valid.astype(jnp.float32)
    return (nbr.astype(jnp.int32), d2e, vmask,
            deg.astype(jnp.float32).reshape(N, 1))


NC = 2
NS = 16
NW = NC * NS
GROWS = N * K // NW
GSUB = 128
GITER_SUB = 4
GCHUNK = GSUB * GITER_SUB


def _sc_gather(pre_pad, nbr2):
    mesh = plsc.VectorSubcoreMesh(core_axis_name="c", subcore_axis_name="s",
                                  num_cores=NC, num_subcores=NS)

    half = 2 * GSUB
    nsteps = GROWS // half

    @functools.partial(
        pl.kernel,
        out_type=jax.ShapeDtypeStruct((N * K, H), jnp.float32),
        mesh=mesh,
        scratch_types=[
            pltpu.VMEM((2, GSUB), jnp.int32),
            pltpu.VMEM((2, GSUB), jnp.int32),
            pltpu.VMEM((half, H), jnp.float32),
            pltpu.VMEM((half, H), jnp.float32),
            pltpu.SemaphoreType.DMA,
            pltpu.SemaphoreType.DMA,
            pltpu.SemaphoreType.DMA,
        ],
    )
    def k(pre_hbm, nbr_hbm, out_hbm, idx_a, idx_b, rows_a, rows_b, gsem, wsem_a, wsem_b):
        wid = lax.axis_index("s") * NC + lax.axis_index("c")
        nbr_row0 = wid * (GROWS // GSUB)
        out_row0 = wid * GROWS

        def body(i, carry):
            for h, (idxv, rowsv, wsem) in enumerate(
                    ((idx_a, rows_a, wsem_a), (idx_b, rows_b, wsem_b))):
                s = i * 2 + h
                @pl.when(s >= 2)
                def _():
                    pltpu.make_async_copy(out_hbm.at[pl.ds(0, half)], rowsv, wsem).wait()
                pltpu.sync_copy(nbr_hbm.at[pl.ds(nbr_row0 + s * 2, 2)], idxv)
                c0 = pltpu.async_copy(pre_hbm.at[idxv.at[0]],
                                      rowsv.at[pl.ds(0, GSUB)], gsem)
                c1 = pltpu.async_copy(pre_hbm.at[idxv.at[1]],
                                      rowsv.at[pl.ds(GSUB, GSUB)], gsem)
                c0.wait()
                c1.wait()
                pltpu.async_copy(rowsv, out_hbm.at[pl.ds(out_row0 + s * half, half)],
                                 wsem)
            return carry

        lax.fori_loop(0, nsteps // 2, body, 0)
        pltpu.make_async_copy(out_hbm.at[pl.ds(0, half)], rows_a, wsem_a).wait()
        pltpu.make_async_copy(out_hbm.at[pl.ds(0, half)], rows_b, wsem_b).wait()

    return k(pre_pad, nbr2)


def kernel(atomic_numbers, positions, batch, emb, msg_W1, msg_b1, msg_W2, msg_b2,
           gru_Wih, gru_Whh, gru_bih, gru_bhh, ln_g, ln_b,
           head_W1, head_b1, head_W2, head_b2):
    emb_pad = jnp.zeros((VMAX_PAD, H), jnp.float32).at[:VMAX].set(emb)
    centers = jnp.full((NRBF_PAD,), 1e6, jnp.float32).at[:NRBF].set(
        jnp.linspace(0.0, CUTOFF, NRBF)).reshape(1, NRBF_PAD)
    W1r_pad = jnp.zeros((L, NRBF_PAD, H), jnp.float32).at[:, :NRBF].set(msg_W1[:, H:])

    x = _embed(atomic_numbers, emb_pad)
    nbr, d2e, vmask, deg = _neighbor_lists(positions)
    nbr2 = nbr.reshape(N * K // GSUB, GSUB)
    for l in range(L):
        pre_pad = _pre(x, msg_W1[l, :H], msg_b1[l])
        G = _sc_gather(pre_pad, nbr2)
        x = _msg_layer(centers, W1r_pad[l], msg_W2[l], msg_b2[l],
                       gru_Wih[l], gru_bih[l], gru_bhh[l], G, d2e, vmask, deg, x)
    return _head(x, batch, ln_g, ln_b, head_W1, head_b1, head_W2, head_b2)
